# Initial kernel scaffold; baseline (speedup 1.0000x reference)
import jax
import jax.numpy as jnp
from jax.experimental import pallas as pl


def kernel(x, token_table, pos_table):
    raise NotImplementedError("write your pallas kernel here")


# speedup vs baseline: 3.3730x; 3.3730x over previous
import functools

import jax
import jax.numpy as jnp
from jax import lax
from jax.experimental import pallas as pl
from jax.experimental.pallas import tpu as pltpu
from jax.experimental.pallas import tpu_sc as plsc

NUM = 100000
EMB_DIM = 64
MAX_NORM = 1.0
BATCH = 4096
SEQ = 200

_INFO = plsc.get_sparse_core_info()
_NW = _INFO.num_cores * _INFO.num_subcores

_R = BATCH * SEQ
_RT = _R // _NW
_SEQ_PER_CHUNK = 4
_CH = _SEQ_PER_CHUNK * SEQ
_NCH = _RT // _CH
_VREGS_PER_SEQ = SEQ * EMB_DIM // 16


def _renorm_body(t_ref, o_ref):
    x = t_ref[...]
    ss = jnp.sum(x * x, axis=1, keepdims=True)
    norm = jnp.sqrt(ss)
    scale = jnp.where(norm > MAX_NORM, MAX_NORM / (norm + 1e-7), 1.0)
    o_ref[...] = x * scale


def _renorm(table, block_rows):
    rows = table.shape[0]
    return pl.pallas_call(
        _renorm_body,
        grid=(rows // block_rows,),
        in_specs=[pl.BlockSpec((block_rows, EMB_DIM), lambda i: (i, 0))],
        out_specs=pl.BlockSpec((block_rows, EMB_DIM), lambda i: (i, 0)),
        out_shape=jax.ShapeDtypeStruct((rows, EMB_DIM), jnp.float32),
    )(table)


def _sc_body(x_hbm, tab_hbm, pos_hbm, out_hbm, idx_v, tok_v, pos_v, sem):
    wid = lax.axis_index("c") * _INFO.num_subcores + lax.axis_index("s")
    pltpu.sync_copy(pos_hbm, pos_v)

    def chunk_body(g, carry):
        base = wid * _RT + g * _CH
        pltpu.sync_copy(x_hbm.at[pl.ds(base, _CH)], idx_v)
        pltpu.async_copy(tab_hbm.at[idx_v], tok_v, sem).wait()

        def add_body(i, carry2):
            for u in range(8):
                v = i * 8 + u
                p = v >> 2
                col = (v & 3) * 16
                pv = pos_v[pl.ds(v * 16, 16)]
                for s in range(_SEQ_PER_CHUNK):
                    plsc.addupdate(tok_v.at[s * SEQ + p, pl.ds(col, 16)], pv)
            return carry2

        lax.fori_loop(0, _VREGS_PER_SEQ // 8, add_body, 0)
        pltpu.sync_copy(tok_v, out_hbm.at[pl.ds(base, _CH)])
        return carry

    lax.fori_loop(0, _NCH, chunk_body, 0)


_sc_lookup = functools.partial(
    pl.kernel,
    mesh=plsc.VectorSubcoreMesh(core_axis_name="c", subcore_axis_name="s"),
    out_type=jax.ShapeDtypeStruct((_R, EMB_DIM), jnp.float32),
    scratch_types=[
        pltpu.VMEM((_CH,), jnp.int32),
        pltpu.VMEM((_CH, EMB_DIM), jnp.float32),
        pltpu.VMEM((SEQ * EMB_DIM,), jnp.float32),
        pltpu.SemaphoreType.DMA,
    ],
    compiler_params=pltpu.CompilerParams(use_tc_tiling_on_sc=False),
)(_sc_body)


def kernel(x, token_table, pos_table):
    scaled_tab = _renorm(token_table, 1000)
    scaled_pos = _renorm(lax.slice(pos_table, (0, 0), (SEQ, EMB_DIM)), SEQ)
    x_flat = x.astype(jnp.int32).reshape(_R)
    out = _sc_lookup(x_flat, scaled_tab, scaled_pos.reshape(SEQ * EMB_DIM))
    return out.reshape(BATCH, SEQ, EMB_DIM)
